# baseline (speedup 1.0000x reference)
import functools

import jax
import jax.numpy as jnp
from jax import lax
from jax.experimental import pallas as pl
from jax.experimental.pallas import tpu as pltpu
from jax.experimental.pallas import tpu_sc as plsc

_LANES = 16


def _add_pos_kernel(x_ref, pos_ref, o_ref):
    o_ref[...] = x_ref[...] + pos_ref[...][None]


def _tc_add(x, pos_embed, H, W, C):
    B = x.shape[0]
    return pl.pallas_call(
        _add_pos_kernel,
        grid=(B,),
        in_specs=[
            pl.BlockSpec((1, H, W, C), lambda b: (b, 0, 0, 0)),
            pl.BlockSpec((H, W, C), lambda b: (0, 0, 0)),
        ],
        out_specs=pl.BlockSpec((1, H, W, C), lambda b: (b, 0, 0, 0)),
        out_shape=jax.ShapeDtypeStruct(x.shape, x.dtype),
        compiler_params=pltpu.CompilerParams(
            dimension_semantics=("arbitrary",),
        ),
    )(x, pos_embed)


def _make_sc_add(Bsc, H, WC, pos_cols):
    info = plsc.get_sparse_core_info()
    NC, NS = info.num_cores, info.num_subcores
    NW = NC * NS
    assert H % NW == 0 and WC % _LANES == 0
    rows_per_w = H // NW
    nvec = WC // _LANES
    mesh = plsc.VectorSubcoreMesh(core_axis_name="c", subcore_axis_name="s")

    @functools.partial(
        pl.kernel,
        out_type=jax.ShapeDtypeStruct((Bsc * H, WC), jnp.float32),
        mesh=mesh,
        scratch_types=[
            pltpu.VMEM((rows_per_w, WC), jnp.float32),
            pltpu.VMEM((Bsc, WC), jnp.float32),
        ],
    )
    def sc_add(x_hbm, pos_hbm, out_hbm, pos_v, xbuf):
        wid = lax.axis_index("s") * NC + lax.axis_index("c")
        base = wid * rows_per_w
        pltpu.sync_copy(
            pos_hbm.at[pl.ds(base, rows_per_w), pl.ds(0, WC)], pos_v
        )
        for h in range(rows_per_w):
            for b in range(Bsc):
                pltpu.sync_copy(
                    x_hbm.at[pl.ds(b * H + base + h, 1), :],
                    xbuf.at[pl.ds(b, 1)],
                )

            def body(j, carry):
                off = pl.ds(j * _LANES, _LANES)
                pv = pos_v[h, off]
                for b in range(Bsc):
                    xbuf[b, off] = xbuf[b, off] + pv
                return carry

            lax.fori_loop(0, nvec, body, jnp.int32(0))
            for b in range(Bsc):
                pltpu.sync_copy(
                    xbuf.at[pl.ds(b, 1)],
                    out_hbm.at[pl.ds(b * H + base + h, 1), :],
                )

    return sc_add


def kernel(x, pos_embed):
    B, H, W, C = x.shape
    B_TC = 14
    Bsc = B - B_TC
    out_tc = _tc_add(x[:B_TC], pos_embed, H, W, C)
    pos2 = pos_embed.reshape(pos_embed.shape[0], -1)
    x2 = x[B_TC:].reshape(Bsc * H, W * C)
    sc_add = _make_sc_add(Bsc, H, W * C, pos2.shape[1])
    out_sc = sc_add(x2, pos2).reshape(Bsc, H, W, C)
    return jnp.concatenate([out_tc, out_sc], axis=0)


# speedup vs baseline: 1.3647x; 1.3647x over previous
import functools

import jax
import jax.numpy as jnp
from jax import lax
from jax.experimental import pallas as pl
from jax.experimental.pallas import tpu as pltpu
from jax.experimental.pallas import tpu_sc as plsc

_LANES = 16


def _add_pos_kernel(x_ref, pos_ref, o_ref):
    o_ref[...] = x_ref[...] + pos_ref[...][None]


def _tc_add(x, pos_embed, H, W, C):
    B = x.shape[0]
    return pl.pallas_call(
        _add_pos_kernel,
        grid=(B,),
        in_specs=[
            pl.BlockSpec((1, H, W, C), lambda b: (b, 0, 0, 0)),
            pl.BlockSpec((H, W, C), lambda b: (0, 0, 0)),
        ],
        out_specs=pl.BlockSpec((1, H, W, C), lambda b: (b, 0, 0, 0)),
        out_shape=jax.ShapeDtypeStruct(x.shape, x.dtype),
        compiler_params=pltpu.CompilerParams(
            dimension_semantics=("arbitrary",),
        ),
    )(x, pos_embed)


def _make_sc_add(B, B_TC, H, W, C):
    Bsc = B - B_TC
    info = plsc.get_sparse_core_info()
    NC, NS = info.num_cores, info.num_subcores
    NW = NC * NS
    assert H % NW == 0 and C % _LANES == 0
    rows_per_w = H // NW
    cvec = C // _LANES
    mesh = plsc.VectorSubcoreMesh(core_axis_name="c", subcore_axis_name="s")

    @functools.partial(
        pl.kernel,
        out_type=jax.ShapeDtypeStruct((Bsc, H, W, C), jnp.float32),
        mesh=mesh,
        scratch_types=[
            pltpu.VMEM((rows_per_w, W, C), jnp.float32),
            pltpu.VMEM((Bsc, W, C), jnp.float32),
        ],
    )
    def sc_add(x_hbm, pos_hbm, out_hbm, pos_v, xbuf):
        wid = lax.axis_index("s") * NC + lax.axis_index("c")
        base = wid * rows_per_w
        pltpu.sync_copy(
            pos_hbm.at[pl.ds(base, rows_per_w), pl.ds(0, W), :], pos_v
        )
        for h in range(rows_per_w):
            for b in range(Bsc):
                pltpu.sync_copy(
                    x_hbm.at[B_TC + b, base + h], xbuf.at[b]
                )

            def body(w, carry):
                for k in range(cvec):
                    off = pl.ds(k * _LANES, _LANES)
                    pv = pos_v[h, w, off]
                    for b in range(Bsc):
                        xbuf[b, w, off] = xbuf[b, w, off] + pv
                return carry

            lax.fori_loop(0, W, body, jnp.int32(0))
            for b in range(Bsc):
                pltpu.sync_copy(
                    xbuf.at[b], out_hbm.at[b, base + h]
                )

    return sc_add


def kernel(x, pos_embed):
    B, H, W, C = x.shape
    B_TC = 14
    out_tc = _tc_add(x[:B_TC], pos_embed, H, W, C)
    sc_add = _make_sc_add(B, B_TC, H, W, C)
    out_sc = sc_add(x, pos_embed)
    return jnp.concatenate([out_tc, out_sc], axis=0)


# speedup vs baseline: 4.1710x; 3.0563x over previous
import jax
import jax.numpy as jnp
from jax.experimental import pallas as pl
from jax.experimental.pallas import tpu as pltpu


def _add_pos_kernel(x_ref, pos_ref, o_ref):
    o_ref[...] = x_ref[...] + pos_ref[...][None]


def kernel(x, pos_embed):
    B, H, W, C = x.shape
    out = pl.pallas_call(
        _add_pos_kernel,
        grid=(B,),
        in_specs=[
            pl.BlockSpec((1, H, W, C), lambda b: (b, 0, 0, 0)),
            pl.BlockSpec((H, W, C), lambda b: (0, 0, 0)),
        ],
        out_specs=pl.BlockSpec((1, H, W, C), lambda b: (b, 0, 0, 0)),
        out_shape=jax.ShapeDtypeStruct(x.shape, x.dtype),
        compiler_params=pltpu.CompilerParams(
            dimension_semantics=("arbitrary",),
        ),
    )(x, pos_embed)
    return out


# speedup vs baseline: 4.1845x; 1.0032x over previous
import jax
import jax.numpy as jnp
from jax.experimental import pallas as pl
from jax.experimental.pallas import tpu as pltpu

_NBUF = 2


def _make_body(B, H, W, C):
    def body(x_hbm, pos_hbm, o_hbm, xbuf, obuf, pos_v, insem, outsem, possem):
        pos_cp = pltpu.make_async_copy(
            pos_hbm.at[pl.ds(0, H), pl.ds(0, W), :], pos_v, possem
        )
        pos_cp.start()

        def in_cp(b, slot):
            return pltpu.make_async_copy(
                x_hbm.at[b], xbuf.at[slot], insem.at[slot]
            )

        def out_cp(b, slot):
            return pltpu.make_async_copy(
                obuf.at[slot], o_hbm.at[b], outsem.at[slot]
            )

        for b in range(_NBUF):
            in_cp(b, b).start()
        pos_cp.wait()
        for b in range(B):
            slot = b % _NBUF
            in_cp(b, slot).wait()
            if b >= _NBUF:
                out_cp(b - _NBUF, slot).wait()
            obuf[slot] = xbuf[slot] + pos_v[...]
            if b + _NBUF < B:
                in_cp(b + _NBUF, slot).start()
            out_cp(b, slot).start()
        for b in range(B - _NBUF, B):
            out_cp(b, b % _NBUF).wait()

    return body


def kernel(x, pos_embed):
    B, H, W, C = x.shape
    out = pl.pallas_call(
        _make_body(B, H, W, C),
        in_specs=[
            pl.BlockSpec(memory_space=pltpu.MemorySpace.HBM),
            pl.BlockSpec(memory_space=pltpu.MemorySpace.HBM),
        ],
        out_specs=pl.BlockSpec(memory_space=pltpu.MemorySpace.HBM),
        out_shape=jax.ShapeDtypeStruct(x.shape, x.dtype),
        scratch_shapes=[
            pltpu.VMEM((_NBUF, H, W, C), jnp.float32),
            pltpu.VMEM((_NBUF, H, W, C), jnp.float32),
            pltpu.VMEM((H, W, C), jnp.float32),
            pltpu.SemaphoreType.DMA((_NBUF,)),
            pltpu.SemaphoreType.DMA((_NBUF,)),
            pltpu.SemaphoreType.DMA,
        ],
    )(x, pos_embed)
    return out


# speedup vs baseline: 4.1948x; 1.0025x over previous
import jax
import jax.numpy as jnp
from jax.experimental import pallas as pl
from jax.experimental.pallas import tpu as pltpu

_NBUF = 4
_CH = 2


def _make_body(B, H, W, C):
    HH = H // _CH
    NCHUNK = B * _CH

    def body(x_hbm, pos_hbm, o_hbm, xbuf, obuf, pos_v, insem, outsem, possem):
        pos_cp = pltpu.make_async_copy(
            pos_hbm.at[pl.ds(0, H), pl.ds(0, W), :], pos_v, possem
        )
        pos_cp.start()

        def in_cp(k, slot):
            b, half = divmod(k, _CH)
            return pltpu.make_async_copy(
                x_hbm.at[b, pl.ds(half * HH, HH)], xbuf.at[slot],
                insem.at[slot],
            )

        def out_cp(k, slot):
            b, half = divmod(k, _CH)
            return pltpu.make_async_copy(
                obuf.at[slot], o_hbm.at[b, pl.ds(half * HH, HH)],
                outsem.at[slot],
            )

        for k in range(_NBUF):
            in_cp(k, k).start()
        pos_cp.wait()
        for k in range(NCHUNK):
            slot = k % _NBUF
            half = k % _CH
            in_cp(k, slot).wait()
            if k >= _NBUF:
                out_cp(k - _NBUF, slot).wait()
            obuf[slot] = xbuf[slot] + pos_v[pl.ds(half * HH, HH)]
            if k + _NBUF < NCHUNK:
                in_cp(k + _NBUF, slot).start()
            out_cp(k, slot).start()
        for k in range(NCHUNK - _NBUF, NCHUNK):
            out_cp(k, k % _NBUF).wait()

    return body


def kernel(x, pos_embed):
    B, H, W, C = x.shape
    HH = H // _CH
    out = pl.pallas_call(
        _make_body(B, H, W, C),
        in_specs=[
            pl.BlockSpec(memory_space=pltpu.MemorySpace.HBM),
            pl.BlockSpec(memory_space=pltpu.MemorySpace.HBM),
        ],
        out_specs=pl.BlockSpec(memory_space=pltpu.MemorySpace.HBM),
        out_shape=jax.ShapeDtypeStruct(x.shape, x.dtype),
        scratch_shapes=[
            pltpu.VMEM((_NBUF, HH, W, C), jnp.float32),
            pltpu.VMEM((_NBUF, HH, W, C), jnp.float32),
            pltpu.VMEM((H, W, C), jnp.float32),
            pltpu.SemaphoreType.DMA((_NBUF,)),
            pltpu.SemaphoreType.DMA((_NBUF,)),
            pltpu.SemaphoreType.DMA,
        ],
    )(x, pos_embed)
    return out
